# Initial kernel scaffold; baseline (speedup 1.0000x reference)
import jax
import jax.numpy as jnp
from jax.experimental import pallas as pl


def kernel(desired_content, memory, key_strength, free_gate, read_weighting, previous_usage, write_weighting):
    raise NotImplementedError("write your pallas kernel here")


# speedup vs baseline: 8.5548x; 8.5548x over previous
import jax
import jax.numpy as jnp
from jax import lax
from jax.experimental import pallas as pl
from jax.experimental.pallas import tpu as pltpu

_N = 1000000
_W = 64
_R = 4
_NB = 125
_B = _N // _NB


def _stream_body(d_ref, ks_ref, fg_ref, mem_ref, rw_ref, pu_ref, ww_ref,
                 e_ref, u_ref, esum_ref):
    d = d_ref[...]
    m = mem_ref[...]
    dot = lax.dot_general(d, m, (((1,), (1,)), ((), ())),
                          preferred_element_type=jnp.float32)
    ones = jnp.ones((1, _W), dtype=jnp.float32)
    sumsq = lax.dot_general(ones, m * m, (((1,), (1,)), ((), ())),
                            preferred_element_type=jnp.float32)
    n1 = jnp.sqrt(jnp.sum(d * d))
    sim = dot / jnp.maximum(n1 * jnp.sqrt(sumsq), 1e-8)
    t = sim * ks_ref[0, 0]
    e = jnp.exp(t)
    e_ref[0] = e
    esum_ref[0] = jnp.sum(e).reshape(1, 1)

    rwb = rw_ref[0]
    fgb = fg_ref[...]
    ib = 1.0 - rwb * fgb
    ret = ib[0:1] * ib[1:2] * ib[2:3] * ib[3:4]
    pu = pu_ref[0]
    ww = ww_ref[0]
    u_ref[0] = (pu + ww - pu * ww) * ret


def _select_body(e_ref, u_ref, s_ref, out_ref, scr_ref):
    inv = 1.0 / s_ref[0, 0]
    out_ref[0, :, :] = e_ref[...] * inv
    u = u_ref[...]
    out_ref[1, :, :] = u
    out_ref[2, :, :] = jnp.zeros((_NB, _B), dtype=jnp.float32)
    scr_ref[...] = u

    lin2d = (lax.broadcasted_iota(jnp.int32, (_NB, _B), 0) * _B
             + lax.broadcasted_iota(jnp.int32, (_NB, _B), 1))
    lane = lax.broadcasted_iota(jnp.int32, (1, _B), 1)

    def cond(c):
        j, cp, _ = c
        return (j < _N) & ((j == 0) | (cp > 0.0))

    def body(c):
        j, cp, sp = c
        uu = scr_ref[...]
        mval = jnp.min(uu)
        linidx = jnp.min(jnp.where(uu == mval, lin2d, jnp.int32(2147483647)))
        r = linidx // _B
        l = linidx - r * _B
        val = jnp.where(j == 0, 1.0 - mval, (1.0 - sp) * cp)
        arow = out_ref[2, pl.ds(r, 1), :]
        out_ref[2, pl.ds(r, 1), :] = jnp.where(lane == l, val, arow)
        urow = scr_ref[pl.ds(r, 1), :]
        scr_ref[pl.ds(r, 1), :] = jnp.where(lane == l, 2.0, urow)
        cp2 = jnp.where(j == 0, mval, cp * mval)
        return (j + 1, cp2, mval)

    lax.while_loop(cond, body, (jnp.int32(0), jnp.float32(1.0),
                                jnp.float32(0.0)))


def kernel(desired_content, memory, key_strength, free_gate, read_weighting,
           previous_usage, write_weighting):
    d = desired_content.reshape(1, _W)
    ks = key_strength.reshape(1, 1)
    fg = free_gate.reshape(_R, 1)
    rw3 = read_weighting.reshape(_NB, _B, _R).transpose(0, 2, 1)
    pu3 = previous_usage.reshape(_NB, 1, _B)
    ww3 = write_weighting.reshape(_NB, 1, _B)

    e3, u3, esum = pl.pallas_call(
        _stream_body,
        grid=(_NB,),
        in_specs=[
            pl.BlockSpec((1, _W), lambda i: (0, 0)),
            pl.BlockSpec((1, 1), lambda i: (0, 0)),
            pl.BlockSpec((_R, 1), lambda i: (0, 0)),
            pl.BlockSpec((_B, _W), lambda i: (i, 0)),
            pl.BlockSpec((1, _R, _B), lambda i: (i, 0, 0)),
            pl.BlockSpec((1, 1, _B), lambda i: (i, 0, 0)),
            pl.BlockSpec((1, 1, _B), lambda i: (i, 0, 0)),
        ],
        out_specs=[
            pl.BlockSpec((1, 1, _B), lambda i: (i, 0, 0)),
            pl.BlockSpec((1, 1, _B), lambda i: (i, 0, 0)),
            pl.BlockSpec((1, 1, 1), lambda i: (i, 0, 0)),
        ],
        out_shape=[
            jax.ShapeDtypeStruct((_NB, 1, _B), jnp.float32),
            jax.ShapeDtypeStruct((_NB, 1, _B), jnp.float32),
            jax.ShapeDtypeStruct((_NB, 1, 1), jnp.float32),
        ],
    )(d, ks, fg, memory, rw3, pu3, ww3)

    s = jnp.sum(esum).reshape(1, 1)

    out2 = pl.pallas_call(
        _select_body,
        in_specs=[
            pl.BlockSpec((_NB, _B), lambda: (0, 0)),
            pl.BlockSpec((_NB, _B), lambda: (0, 0)),
            pl.BlockSpec((1, 1), lambda: (0, 0)),
        ],
        out_specs=pl.BlockSpec((3, _NB, _B), lambda: (0, 0, 0)),
        out_shape=jax.ShapeDtypeStruct((3, _NB, _B), jnp.float32),
        scratch_shapes=[pltpu.VMEM((_NB, _B), jnp.float32)],
    )(e3.reshape(_NB, _B), u3.reshape(_NB, _B), s)

    return out2.reshape(3, _N)


# speedup vs baseline: 8.7241x; 1.0198x over previous
import jax
import jax.numpy as jnp
from jax import lax
from jax.experimental import pallas as pl
from jax.experimental.pallas import tpu as pltpu

_N = 1000000
_W = 64
_R = 4
_NB = 125
_B = _N // _NB


def _stream_body(d_ref, ks_ref, fg_ref, mem_ref, rw_ref, pu_ref, ww_ref,
                 e_ref, u_ref, esum_ref, umin_ref):
    d = d_ref[...]
    m = mem_ref[...]
    dot = lax.dot_general(d, m, (((1,), (1,)), ((), ())),
                          preferred_element_type=jnp.float32)
    ones = jnp.ones((1, _W), dtype=jnp.float32)
    sumsq = lax.dot_general(ones, m * m, (((1,), (1,)), ((), ())),
                            preferred_element_type=jnp.float32)
    n1 = jnp.sqrt(jnp.sum(d * d))
    sim = dot / jnp.maximum(n1 * jnp.sqrt(sumsq), 1e-8)
    t = sim * ks_ref[0, 0]
    e = jnp.exp(t)
    e_ref[0] = e
    esum_ref[0] = jnp.sum(e).reshape(1, 1)

    rwb = rw_ref[0]
    fgb = fg_ref[...]
    ib = 1.0 - rwb * fgb
    ret = ib[0:1] * ib[1:2] * ib[2:3] * ib[3:4]
    pu = pu_ref[0]
    ww = ww_ref[0]
    u = (pu + ww - pu * ww) * ret
    u_ref[0] = u
    umin_ref[0] = jnp.min(u).reshape(1, 1)


def _select_body(e_ref, u_ref, rm0_ref, s_ref, out_ref, scr_ref, rm_ref):
    inv = 1.0 / s_ref[0, 0]
    out_ref[0, :, :] = e_ref[...] * inv
    u = u_ref[...]
    out_ref[1, :, :] = u
    out_ref[2, :, :] = jnp.zeros((_NB, _B), dtype=jnp.float32)
    scr_ref[...] = u
    rm_ref[...] = rm0_ref[...]

    riota = lax.broadcasted_iota(jnp.int32, (_NB, 1), 0)
    lane = lax.broadcasted_iota(jnp.int32, (1, _B), 1)
    big = jnp.int32(2147483647)

    def cond(c):
        j, cp, _ = c
        return (j < _N) & ((j == 0) | (cp > 0.0))

    def body(c):
        j, cp, sp = c
        rm = rm_ref[...]
        mval = jnp.min(rm)
        r = jnp.min(jnp.where(rm == mval, riota, big))
        row = scr_ref[pl.ds(r, 1), :]
        l = jnp.min(jnp.where(row == mval, lane, big))
        val = jnp.where(j == 0, 1.0 - mval, (1.0 - sp) * cp)
        arow = out_ref[2, pl.ds(r, 1), :]
        out_ref[2, pl.ds(r, 1), :] = jnp.where(lane == l, val, arow)
        newrow = jnp.where(lane == l, 2.0, row)
        scr_ref[pl.ds(r, 1), :] = newrow
        rm_ref[pl.ds(r, 1), :] = jnp.min(newrow).reshape(1, 1)
        cp2 = jnp.where(j == 0, mval, cp * mval)
        return (j + 1, cp2, mval)

    lax.while_loop(cond, body, (jnp.int32(0), jnp.float32(1.0),
                                jnp.float32(0.0)))


def kernel(desired_content, memory, key_strength, free_gate, read_weighting,
           previous_usage, write_weighting):
    d = desired_content.reshape(1, _W)
    ks = key_strength.reshape(1, 1)
    fg = free_gate.reshape(_R, 1)
    rw3 = read_weighting.reshape(_NB, _B, _R).transpose(0, 2, 1)
    pu3 = previous_usage.reshape(_NB, 1, _B)
    ww3 = write_weighting.reshape(_NB, 1, _B)

    e3, u3, esum, umin = pl.pallas_call(
        _stream_body,
        grid=(_NB,),
        in_specs=[
            pl.BlockSpec((1, _W), lambda i: (0, 0)),
            pl.BlockSpec((1, 1), lambda i: (0, 0)),
            pl.BlockSpec((_R, 1), lambda i: (0, 0)),
            pl.BlockSpec((_B, _W), lambda i: (i, 0)),
            pl.BlockSpec((1, _R, _B), lambda i: (i, 0, 0)),
            pl.BlockSpec((1, 1, _B), lambda i: (i, 0, 0)),
            pl.BlockSpec((1, 1, _B), lambda i: (i, 0, 0)),
        ],
        out_specs=[
            pl.BlockSpec((1, 1, _B), lambda i: (i, 0, 0)),
            pl.BlockSpec((1, 1, _B), lambda i: (i, 0, 0)),
            pl.BlockSpec((1, 1, 1), lambda i: (i, 0, 0)),
            pl.BlockSpec((1, 1, 1), lambda i: (i, 0, 0)),
        ],
        out_shape=[
            jax.ShapeDtypeStruct((_NB, 1, _B), jnp.float32),
            jax.ShapeDtypeStruct((_NB, 1, _B), jnp.float32),
            jax.ShapeDtypeStruct((_NB, 1, 1), jnp.float32),
            jax.ShapeDtypeStruct((_NB, 1, 1), jnp.float32),
        ],
    )(d, ks, fg, memory, rw3, pu3, ww3)

    s = jnp.sum(esum).reshape(1, 1)

    out2 = pl.pallas_call(
        _select_body,
        in_specs=[
            pl.BlockSpec((_NB, _B), lambda: (0, 0)),
            pl.BlockSpec((_NB, _B), lambda: (0, 0)),
            pl.BlockSpec((_NB, 1), lambda: (0, 0)),
            pl.BlockSpec((1, 1), lambda: (0, 0)),
        ],
        out_specs=pl.BlockSpec((3, _NB, _B), lambda: (0, 0, 0)),
        out_shape=jax.ShapeDtypeStruct((3, _NB, _B), jnp.float32),
        scratch_shapes=[pltpu.VMEM((_NB, _B), jnp.float32),
                        pltpu.VMEM((_NB, 1), jnp.float32)],
    )(e3.reshape(_NB, _B), u3.reshape(_NB, _B), umin.reshape(_NB, 1), s)

    return out2.reshape(3, _N)


# speedup vs baseline: 8.9208x; 1.0225x over previous
import jax
import jax.numpy as jnp
from jax import lax
from jax.experimental import pallas as pl
from jax.experimental.pallas import tpu as pltpu

_N = 1000000
_W = 64
_R = 4
_NB = 125
_C = 1000
_SR = 8
_B = _SR * _C


def _stream_body(d_ref, ks_ref, fg_ref, mem_ref, rw_ref, pu_ref, ww_ref,
                 e_ref, u_ref, esum_ref, umin_ref):
    i = pl.program_id(0)
    d = d_ref[...]
    ones = jnp.ones((1, _W), dtype=jnp.float32)
    lhs = jnp.concatenate([d, ones], axis=0)
    m = mem_ref[...]
    dots = []
    sqs = []
    for g in range(_SR):
        mg = m[g * _C:(g + 1) * _C]
        rg = lax.dot_general(lhs, mg, (((1,), (1,)), ((), ())),
                             preferred_element_type=jnp.float32)
        sg = lax.dot_general(ones, mg * mg, (((1,), (1,)), ((), ())),
                             preferred_element_type=jnp.float32)
        dots.append(rg[0:1])
        sqs.append(sg)
    dot = jnp.concatenate(dots, axis=0)
    sumsq = jnp.concatenate(sqs, axis=0)
    n1 = jnp.sqrt(jnp.sum(d * d))
    sim = dot / jnp.maximum(n1 * jnp.sqrt(sumsq), 1e-8)
    t = sim * ks_ref[0, 0]
    e = jnp.exp(t)
    e_ref[...] = e

    rwb = rw_ref[...]
    fgb = fg_ref[...].reshape(1, _R, 1)
    ib = 1.0 - rwb * fgb
    ret = ib[:, 0, :] * ib[:, 1, :] * ib[:, 2, :] * ib[:, 3, :]
    pu = pu_ref[...]
    ww = ww_ref[...]
    u = (pu + ww - pu * ww) * ret
    u_ref[...] = u

    lane = lax.broadcasted_iota(jnp.int32, (1, 128), 1)

    @pl.when(i == 0)
    def _():
        esum_ref[...] = jnp.zeros((1, 128), jnp.float32)
        umin_ref[...] = jnp.full((1, 128), 2.0, jnp.float32)

    esum_ref[...] = esum_ref[...] + jnp.where(lane == i, jnp.sum(e), 0.0)
    umin_ref[...] = jnp.minimum(umin_ref[...],
                                jnp.where(lane == i, jnp.min(u), 2.0))


def _select_body(e_ref, u_ref, rm0_ref, s_ref, out_ref, scr_ref, rm_ref):
    inv = 1.0 / s_ref[0, 0]
    out_ref[0, :, :] = e_ref[...] * inv
    u = u_ref[...]
    out_ref[1, :, :] = u
    out_ref[2, :, :] = jnp.zeros((_C, _C), dtype=jnp.float32)
    scr_ref[...] = u
    rm_ref[...] = rm0_ref[...]

    lane128 = lax.broadcasted_iota(jnp.int32, (1, 128), 1)
    lin8 = (lax.broadcasted_iota(jnp.int32, (_SR, _C), 0) * _C
            + lax.broadcasted_iota(jnp.int32, (_SR, _C), 1))
    lane1k = lax.broadcasted_iota(jnp.int32, (1, _C), 1)
    big = jnp.int32(2147483647)

    def cond(c):
        j, cp, _ = c
        return (j < _N) & ((j == 0) | (cp > 0.0))

    def body(c):
        j, cp, sp = c
        rm = rm_ref[...]
        mval = jnp.min(rm)
        g = jnp.min(jnp.where(rm == mval, lane128, big))
        sub = scr_ref[pl.ds(g * _SR, _SR), :]
        lin = jnp.min(jnp.where(sub == mval, lin8, big))
        r = g * _SR + lin // _C
        l = lin - (lin // _C) * _C
        val = jnp.where(j == 0, 1.0 - mval, (1.0 - sp) * cp)
        arow = out_ref[2, pl.ds(r, 1), :]
        out_ref[2, pl.ds(r, 1), :] = jnp.where(lane1k == l, val, arow)
        urow = scr_ref[pl.ds(r, 1), :]
        scr_ref[pl.ds(r, 1), :] = jnp.where(lane1k == l, 2.0, urow)
        newsub = scr_ref[pl.ds(g * _SR, _SR), :]
        rm_ref[...] = jnp.where(lane128 == g, jnp.min(newsub), rm)
        cp2 = jnp.where(j == 0, mval, cp * mval)
        return (j + 1, cp2, mval)

    lax.while_loop(cond, body, (jnp.int32(0), jnp.float32(1.0),
                                jnp.float32(0.0)))


def kernel(desired_content, memory, key_strength, free_gate, read_weighting,
           previous_usage, write_weighting):
    d = desired_content.reshape(1, _W)
    ks = key_strength.reshape(1, 1)
    fg = free_gate.reshape(_R, 1)
    rw3 = read_weighting.reshape(_C, _C, _R).transpose(0, 2, 1)
    pu2 = previous_usage.reshape(_C, _C)
    ww2 = write_weighting.reshape(_C, _C)

    e2, u2, esum, umin = pl.pallas_call(
        _stream_body,
        grid=(_NB,),
        in_specs=[
            pl.BlockSpec((1, _W), lambda i: (0, 0)),
            pl.BlockSpec((1, 1), lambda i: (0, 0)),
            pl.BlockSpec((_R, 1), lambda i: (0, 0)),
            pl.BlockSpec((_B, _W), lambda i: (i, 0)),
            pl.BlockSpec((_SR, _R, _C), lambda i: (i, 0, 0)),
            pl.BlockSpec((_SR, _C), lambda i: (i, 0)),
            pl.BlockSpec((_SR, _C), lambda i: (i, 0)),
        ],
        out_specs=[
            pl.BlockSpec((_SR, _C), lambda i: (i, 0)),
            pl.BlockSpec((_SR, _C), lambda i: (i, 0)),
            pl.BlockSpec((1, 128), lambda i: (0, 0)),
            pl.BlockSpec((1, 128), lambda i: (0, 0)),
        ],
        out_shape=[
            jax.ShapeDtypeStruct((_C, _C), jnp.float32),
            jax.ShapeDtypeStruct((_C, _C), jnp.float32),
            jax.ShapeDtypeStruct((1, 128), jnp.float32),
            jax.ShapeDtypeStruct((1, 128), jnp.float32),
        ],
    )(d, ks, fg, memory, rw3, pu2, ww2)

    s = jnp.sum(esum).reshape(1, 1)

    out3 = pl.pallas_call(
        _select_body,
        in_specs=[
            pl.BlockSpec((_C, _C), lambda: (0, 0)),
            pl.BlockSpec((_C, _C), lambda: (0, 0)),
            pl.BlockSpec((1, 128), lambda: (0, 0)),
            pl.BlockSpec((1, 1), lambda: (0, 0)),
        ],
        out_specs=pl.BlockSpec((3, _C, _C), lambda: (0, 0, 0)),
        out_shape=jax.ShapeDtypeStruct((3, _C, _C), jnp.float32),
        scratch_shapes=[pltpu.VMEM((_C, _C), jnp.float32),
                        pltpu.VMEM((1, 128), jnp.float32)],
    )(e2, u2, umin, s)

    return out3.reshape(3, _N)
